# baseline (speedup 1.0000x reference)
import jax
import jax.numpy as jnp
from jax import lax
from jax.experimental import pallas as pl
from jax.experimental.pallas import tpu as pltpu
from jax.experimental.pallas import tpu_sc as plsc

HIDDEN = 256
NVOC = 5
NAUX = 60
NC, NS = 2, 16
CHUNK = 128
TAIL0 = 44


def _sc_body(ids_hbm, aux_hbm, out_hbm, ids_v, aux_v, out_v):
    wid = lax.axis_index("s") * NC + lax.axis_index("c")
    iota = lax.broadcasted_iota(jnp.int32, (16,), 0)
    zeros16 = jnp.zeros((16,), jnp.float32)

    base = 8 * (wid // 4)
    pltpu.sync_copy(ids_hbm.at[pl.ds(base, 8), :], ids_v)

    def zrow(r, carry):
        for k in range(HIDDEN // 16):
            out_v[0, r, pl.ds(k * 16, 16)] = zeros16
        return carry

    lax.fori_loop(0, CHUNK, zrow, 0)

    def jrow_make(row_local, t0):
      def jrow(j, carry):
        idv = ids_v[row_local, pl.ds(t0 + j * 16, 16)]
        lt5f = (lax.shift_right_logical(iota - NVOC, 31)).astype(jnp.float32)
        for l in range(16):
            r = j * 16 + l
            idx = idv[l]
            lt = lax.shift_right_logical(idx - NVOC, 31)
            eq = 1 - jnp.minimum(jnp.abs(iota - idx), 1)
            oh = (eq * lt).astype(jnp.float32)
            out_v[0, r, pl.ds(NVOC + TAIL0, 16)] = aux_v[0, r, pl.ds(TAIL0, 16)]
            for k in (2, 1, 0):
                out_v[0, r, pl.ds(NVOC + k * 16, 16)] = aux_v[0, r, pl.ds(k * 16, 16)]
            w0 = out_v[0, r, pl.ds(0, 16)]
            out_v[0, r, pl.ds(0, 16)] = oh * lt5f + w0 * (1.0 - lt5f)
        return carry

      return jrow

    for ci in range(8):
        b = 2 * wid + ci // 4
        row_local = b - base
        t0 = (ci % 4) * CHUNK
        pltpu.sync_copy(aux_hbm.at[pl.ds(b, 1), pl.ds(t0, CHUNK), :], aux_v)
        lax.fori_loop(0, CHUNK // 16, jrow_make(row_local, t0), 0)
        pltpu.sync_copy(out_v, out_hbm.at[pl.ds(b, 1), pl.ds(t0, CHUNK), :])


def kernel(input_ids, aux_features):
    b, s = input_ids.shape
    mesh = plsc.VectorSubcoreMesh(core_axis_name="c", subcore_axis_name="s")
    f = pl.kernel(
        _sc_body,
        out_type=jax.ShapeDtypeStruct((b, s, HIDDEN), jnp.float32),
        mesh=mesh,
        scratch_types=[
            pltpu.VMEM((8, 512), jnp.int32),
            pltpu.VMEM((1, CHUNK, NAUX), jnp.float32),
            pltpu.VMEM((1, CHUNK, HIDDEN), jnp.float32),
        ],
    )
    return f(input_ids, aux_features)


# speedup vs baseline: 1.2824x; 1.2824x over previous
import jax
import jax.numpy as jnp
from jax import lax
from jax.experimental import pallas as pl
from jax.experimental.pallas import tpu as pltpu
from jax.experimental.pallas import tpu_sc as plsc

HIDDEN = 256
NVOC = 5
NAUX = 60
NC, NS = 2, 16
CHUNK = 128
TAIL0 = 44
NCHUNK = 8


def _sc_body(ids_hbm, aux_hbm, out_hbm, ids_v, aux_v, out_v, aux_sem, out_sem):
    wid = lax.axis_index("s") * NC + lax.axis_index("c")
    iota = lax.broadcasted_iota(jnp.int32, (16,), 0)
    zeros16 = jnp.zeros((16,), jnp.float32)
    lt5f = (lax.shift_right_logical(iota - NVOC, 31)).astype(jnp.float32)

    base = 8 * (wid // 4)
    pltpu.sync_copy(ids_hbm.at[pl.ds(base, 8), :], ids_v)

    def zrow(r, carry):
        for sl in range(2):
            for k in range(HIDDEN // 16):
                out_v[sl, 0, r, pl.ds(k * 16, 16)] = zeros16
        return carry

    lax.fori_loop(0, CHUNK, zrow, 0)

    def bt(ci):
        return 2 * wid + ci // 4, (ci % 4) * CHUNK

    def jrow_make(sl, row_local, t0):
        def jrow(j, carry):
            idv = ids_v[row_local, pl.ds(t0 + j * 16, 16)]
            for l in range(16):
                r = j * 16 + l
                idx = idv[l]
                lt = lax.shift_right_logical(idx - NVOC, 31)
                eq = 1 - jnp.minimum(jnp.abs(iota - idx), 1)
                oh = (eq * lt).astype(jnp.float32)
                out_v[sl, 0, r, pl.ds(NVOC + TAIL0, 16)] = aux_v[
                    sl, 0, r, pl.ds(TAIL0, 16)
                ]
                for k in (2, 1, 0):
                    out_v[sl, 0, r, pl.ds(NVOC + k * 16, 16)] = aux_v[
                        sl, 0, r, pl.ds(k * 16, 16)
                    ]
                w0 = out_v[sl, 0, r, pl.ds(0, 16)]
                out_v[sl, 0, r, pl.ds(0, 16)] = oh * lt5f + w0 * (1.0 - lt5f)
            return carry

        return jrow

    def start_aux(ci):
        b, t0 = bt(ci)
        sl = ci % 2
        return pltpu.async_copy(
            aux_hbm.at[pl.ds(b, 1), pl.ds(t0, CHUNK), :],
            aux_v.at[sl],
            aux_sem.at[sl],
        )

    aux_d = {0: start_aux(0)}
    out_d = {}
    for ci in range(NCHUNK):
        sl = ci % 2
        b, t0 = bt(ci)
        if ci + 1 < NCHUNK:
            aux_d[ci + 1] = start_aux(ci + 1)
        if ci >= 2:
            out_d[ci - 2].wait()
        aux_d[ci].wait()
        lax.fori_loop(0, CHUNK // 16, jrow_make(sl, b - base, t0), 0)
        out_d[ci] = pltpu.async_copy(
            out_v.at[sl],
            out_hbm.at[pl.ds(b, 1), pl.ds(t0, CHUNK), :],
            out_sem.at[sl],
        )
    out_d[NCHUNK - 2].wait()
    out_d[NCHUNK - 1].wait()


def kernel(input_ids, aux_features):
    b, s = input_ids.shape
    mesh = plsc.VectorSubcoreMesh(core_axis_name="c", subcore_axis_name="s")
    f = pl.kernel(
        _sc_body,
        out_type=jax.ShapeDtypeStruct((b, s, HIDDEN), jnp.float32),
        mesh=mesh,
        scratch_types=[
            pltpu.VMEM((8, 512), jnp.int32),
            pltpu.VMEM((2, 1, CHUNK, NAUX), jnp.float32),
            pltpu.VMEM((2, 1, CHUNK, HIDDEN), jnp.float32),
            pltpu.SemaphoreType.DMA((2,)),
            pltpu.SemaphoreType.DMA((2,)),
        ],
    )
    return f(input_ids, aux_features)
